# baseline (speedup 1.0000x reference)
import functools

import jax
import jax.numpy as jnp
from jax import lax
from jax.experimental import pallas as pl
from jax.experimental.pallas import tpu as pltpu
from jax.experimental.pallas import tpu_sc as plsc

N = 10000
E = 320000
D_IN = 128
HEADS = 8
C = 32
HID = 256
NUM_CLASSES = 6

Q_ROW = 80
QC = 2 * C
CHUNK = 80
NCHUNK = 5
K = CHUNK * NCHUNK
NTILES = 16
E_PER_TILE = E // NTILES
NBATCH = E_PER_TILE // K
ROWS_PER_TILE = 632
ACC_N = NTILES * ROWS_PER_TILE
ZROWS = 8

BA = 400
BC = 400


def _proj_body(x_ref, w_ref, as_ref, ad_ref,
               hs0_ref, hs1_ref, hs2_ref, hs3_ref, adout_ref, m_ref,
               ms_acc, md_acc):
    i = pl.program_id(0)
    xb = x_ref[...]
    h = jnp.dot(xb, w_ref[...], preferred_element_type=jnp.float32)
    a_s = jnp.dot(h, as_ref[...], preferred_element_type=jnp.float32)
    a_d = jnp.dot(h, ad_ref[...], preferred_element_type=jnp.float32)
    pad = jnp.zeros((BA, Q_ROW - QC - 2), jnp.float32)
    for q, hs_ref in enumerate([hs0_ref, hs1_ref, hs2_ref, hs3_ref]):
        hs_ref[...] = jnp.concatenate(
            [h[:, q * QC:(q + 1) * QC], a_s[:, 2 * q:2 * q + 2], pad], axis=1)
    adout_ref[...] = a_d
    bm_s = jnp.max(a_s, axis=0, keepdims=True)
    bm_d = jnp.max(a_d, axis=0, keepdims=True)

    @pl.when(i == 0)
    def _():
        ms_acc[...] = bm_s
        md_acc[...] = bm_d

    @pl.when(i > 0)
    def _():
        ms_acc[...] = jnp.maximum(ms_acc[...], bm_s)
        md_acc[...] = jnp.maximum(md_acc[...], bm_d)

    @pl.when(i == pl.num_programs(0) - 1)
    def _():
        m_ref[...] = ms_acc[...] + md_acc[...]


def _project(x, W_gat, As_mat, Ad_mat):
    hs_spec = pl.BlockSpec((BA, Q_ROW), lambda i: (i, 0))
    hs_shape = jax.ShapeDtypeStruct((N, Q_ROW), jnp.float32)
    return pl.pallas_call(
        _proj_body,
        grid=(N // BA,),
        in_specs=[
            pl.BlockSpec((BA, D_IN), lambda i: (i, 0)),
            pl.BlockSpec((D_IN, HEADS * C), lambda i: (0, 0)),
            pl.BlockSpec((HEADS * C, HEADS), lambda i: (0, 0)),
            pl.BlockSpec((HEADS * C, HEADS), lambda i: (0, 0)),
        ],
        out_specs=[
            hs_spec, hs_spec, hs_spec, hs_spec,
            pl.BlockSpec((BA, HEADS), lambda i: (i, 0)),
            pl.BlockSpec((1, HEADS), lambda i: (0, 0)),
        ],
        out_shape=[
            hs_shape, hs_shape, hs_shape, hs_shape,
            jax.ShapeDtypeStruct((N, HEADS), jnp.float32),
            jax.ShapeDtypeStruct((1, HEADS), jnp.float32),
        ],
        scratch_shapes=[
            pltpu.VMEM((1, HEADS), jnp.float32),
            pltpu.VMEM((1, HEADS), jnp.float32),
        ],
    )(x, W_gat, As_mat, Ad_mat)


def _splat(vec, lane):
    return lax.gather(
        vec, jnp.full((16, 1), lane, jnp.int32),
        lax.GatherDimensionNumbers(offset_dims=(), collapsed_slice_dims=(0,),
                                   start_index_map=(0,)),
        (1,), mode=lax.GatherScatterMode.PROMISE_IN_BOUNDS)


@functools.partial(
    pl.kernel,
    mesh=plsc.VectorSubcoreMesh(core_axis_name="c", subcore_axis_name="s"),
    compiler_params=pltpu.CompilerParams(use_tc_tiling_on_sc=False,
                                         needs_layout_passes=False),
    out_type=jax.ShapeDtypeStruct((4 * ACC_N, Q_ROW), jnp.float32),
    scratch_types=[
        pltpu.VMEM((2, NCHUNK, 1, CHUNK), jnp.int32),
        pltpu.VMEM((2, NCHUNK, 1, CHUNK), jnp.int32),
        pltpu.VMEM((2, NCHUNK, 1, CHUNK), jnp.int32),
        pltpu.VMEM((2, K, Q_ROW), jnp.float32),
        pltpu.VMEM((2, K, HEADS), jnp.float32),
        pltpu.VMEM((16,), jnp.float32),
        pltpu.VMEM((ZROWS, Q_ROW), jnp.float32),
        pltpu.VMEM_SHARED((ACC_N, Q_ROW), jnp.float32),
        pltpu.SemaphoreType.DMA((2, NCHUNK)),
        pltpu.SemaphoreType.DMA((2, NCHUNK)),
        pltpu.SemaphoreType.DMA((2, NCHUNK)),
    ],
)
def _sc_edge(src_hbm, dst_hbm, hs_hbm, ad_hbm, m_hbm, out_hbm,
             src_v, dst_v, srcadj_v, rows_v, adv, m_v, zbuf, acc,
             gsem, asem, ssem):
    c = lax.axis_index("c")
    sid = lax.axis_index("s")

    pltpu.sync_copy(m_hbm, m_v)

    def zrow(r, _):
        for k in range(Q_ROW // 16):
            zbuf[r, pl.ds(k * 16, 16)] = jnp.zeros((16,), jnp.float32)
        return 0

    lax.fori_loop(0, ZROWS, zrow, 0)

    iota16 = lax.iota(jnp.int32, 16)

    for half in range(2):
        q = 2 * half + c
        qn = q * N

        def zacc(j, _):
            pltpu.sync_copy(
                zbuf, acc.at[pl.ds(sid * ROWS_PER_TILE + j * ZROWS, ZROWS)])
            return 0

        lax.fori_loop(0, ROWS_PER_TILE // ZROWS, zacc, 0)
        plsc.subcore_barrier()

        def fire_batch(b, buf):
            bidx = sid * NBATCH + b
            pltpu.sync_copy(src_hbm.at[bidx], src_v.at[buf])
            pltpu.sync_copy(dst_hbm.at[bidx], dst_v.at[buf])
            for j in range(NCHUNK):
                for k in range(CHUNK // 16):
                    srcadj_v[buf, j, 0, pl.ds(k * 16, 16)] = (
                        src_v[buf, j, 0, pl.ds(k * 16, 16)] + qn)
            for j in range(NCHUNK):
                pltpu.async_copy(
                    hs_hbm.at[srcadj_v.at[buf, j, 0]],
                    rows_v.at[buf, pl.ds(j * CHUNK, CHUNK)], gsem.at[buf, j])
                pltpu.async_copy(
                    ad_hbm.at[dst_v.at[buf, j, 0]],
                    adv.at[buf, pl.ds(j * CHUNK, CHUNK)], asem.at[buf, j])

        def wait_scatters(buf):
            for j in range(NCHUNK):
                pltpu.make_async_copy(
                    rows_v.at[buf, pl.ds(j * CHUNK, CHUNK)],
                    acc.at[dst_v.at[buf, j, 0]], ssem.at[buf, j]).wait()

        def make_group(buf):
            def group(g, _):
                base = g * 16
                rid = base + iota16
                wv = []
                for hd in range(2):
                    col_w = jnp.full((16,), QC + hd, jnp.int32)
                    a_s = plsc.load_gather(rows_v.at[buf], [rid, col_w])
                    hcol = jnp.zeros((16,), jnp.int32) + (2 * q + hd)
                    a_d = plsc.load_gather(adv.at[buf], [rid, hcol])
                    v = a_s + a_d
                    lr = jnp.where(v >= 0, v, 0.2 * v)
                    w = jnp.exp(lr - plsc.load_gather(m_v, [hcol]))
                    plsc.store_scatter(rows_v.at[buf], [rid, col_w], w)
                    wv.append(w)
                for e in range(16):
                    r = base + e
                    for hd in range(2):
                        ws = _splat(wv[hd], e)
                        for hv in range(2):
                            sl = pl.ds(hd * C + hv * 16, 16)
                            rows_v[buf, r, sl] = rows_v[buf, r, sl] * ws
                return 0

            return group

        def phase(b, buf, obuf):
            group = make_group(buf)
            gpc = CHUNK // 16
            for j in range(NCHUNK):
                pltpu.make_async_copy(
                    hs_hbm.at[srcadj_v.at[buf, j, 0]],
                    rows_v.at[buf, pl.ds(j * CHUNK, CHUNK)],
                    gsem.at[buf, j]).wait()
                pltpu.make_async_copy(
                    ad_hbm.at[dst_v.at[buf, j, 0]],
                    adv.at[buf, pl.ds(j * CHUNK, CHUNK)],
                    asem.at[buf, j]).wait()
                lax.fori_loop(j * gpc, (j + 1) * gpc, group, 0)
                pltpu.async_copy(
                    rows_v.at[buf, pl.ds(j * CHUNK, CHUNK)],
                    acc.at[dst_v.at[buf, j, 0]], ssem.at[buf, j], add=True)

            @pl.when(jnp.logical_and(b >= 1, b + 1 < NBATCH))
            def _():
                wait_scatters(obuf)

            @pl.when(b + 1 < NBATCH)
            def _():
                fire_batch(b + 1, obuf)

        fire_batch(0, 0)

        def two_phases(t, _):
            phase(2 * t, 0, 1)
            phase(2 * t + 1, 1, 0)
            return 0

        lax.fori_loop(0, NBATCH // 2, two_phases, 0)
        wait_scatters(0)
        wait_scatters(1)
        plsc.subcore_barrier()
        pltpu.sync_copy(
            acc.at[pl.ds(sid * ROWS_PER_TILE, ROWS_PER_TILE)],
            out_hbm.at[pl.ds(q * ACC_N + sid * ROWS_PER_TILE,
                             ROWS_PER_TILE)])


def _final_body(hs0_ref, hs1_ref, hs2_ref, hs3_ref,
                n0_ref, n1_ref, n2_ref, n3_ref, ad_ref, m_ref,
                bias_ref, w1_ref, b1_ref, w2_ref, b2_ref, y_ref, acc):
    i = pl.program_id(0)
    ad = ad_ref[...]
    m = m_ref[...]
    ii = lax.broadcasted_iota(jnp.int32, (2, QC), 1) // C
    hh = lax.broadcasted_iota(jnp.int32, (2, QC), 0)
    expand = (ii == hh).astype(jnp.float32)

    outs = []
    for q, (hs_ref, nm_ref) in enumerate([(hs0_ref, n0_ref), (hs1_ref, n1_ref),
                                          (hs2_ref, n2_ref), (hs3_ref, n3_ref)]):
        hs = hs_ref[...]
        nm = nm_ref[...]
        h = hs[:, :QC]
        a_s = hs[:, QC:QC + 2]
        a_d = ad[:, 2 * q:2 * q + 2]
        mm = m[:, 2 * q:2 * q + 2]
        v = a_s + a_d
        lr = jnp.where(v >= 0, v, 0.2 * v)
        ws = jnp.exp(lr - mm)
        den = nm[:, QC:QC + 2] + ws
        ws_x = jnp.dot(ws, expand, preferred_element_type=jnp.float32)
        den_x = jnp.dot(den, expand, preferred_element_type=jnp.float32)
        outs.append((nm[:, :QC] + ws_x * h) / den_x)
    out = jnp.concatenate(outs, axis=1) + bias_ref[...]
    out = jnp.where(out > 0, out, jnp.exp(out) - 1.0)
    psum = jnp.sum(out, axis=0, keepdims=True)

    @pl.when(i == 0)
    def _():
        acc[...] = psum

    @pl.when(i > 0)
    def _():
        acc[...] = acc[...] + psum

    @pl.when(i == pl.num_programs(0) - 1)
    def _():
        pooled = acc[...] * (1.0 / N)
        hmid = jnp.maximum(
            jnp.dot(pooled, w1_ref[...], preferred_element_type=jnp.float32)
            + b1_ref[...], 0.0)
        y_ref[...] = (jnp.dot(hmid, w2_ref[...],
                              preferred_element_type=jnp.float32)
                      + b2_ref[...])


def _finalize(hsq, numq, ad, m, bias_gat, W1, b1, W2, b2):
    hs_spec = pl.BlockSpec((BC, Q_ROW), lambda i: (i, 0))
    return pl.pallas_call(
        _final_body,
        grid=(N // BC,),
        in_specs=[
            hs_spec, hs_spec, hs_spec, hs_spec,
            hs_spec, hs_spec, hs_spec, hs_spec,
            pl.BlockSpec((BC, HEADS), lambda i: (i, 0)),
            pl.BlockSpec((1, HEADS), lambda i: (0, 0)),
            pl.BlockSpec((1, HEADS * C), lambda i: (0, 0)),
            pl.BlockSpec((HID, HID // 2), lambda i: (0, 0)),
            pl.BlockSpec((1, HID // 2), lambda i: (0, 0)),
            pl.BlockSpec((HID // 2, NUM_CLASSES), lambda i: (0, 0)),
            pl.BlockSpec((1, NUM_CLASSES), lambda i: (0, 0)),
        ],
        out_specs=pl.BlockSpec((1, NUM_CLASSES), lambda i: (0, 0)),
        out_shape=jax.ShapeDtypeStruct((1, NUM_CLASSES), jnp.float32),
        scratch_shapes=[pltpu.VMEM((1, HEADS * C), jnp.float32)],
    )(*hsq, *numq, ad, m, bias_gat, W1, b1, W2, b2)


def kernel(x, edge_index, W_gat, att_src, att_dst, bias_gat, W1, b1, W2, b2):
    ii = jnp.arange(HEADS * C)
    heads = jnp.arange(HEADS)
    sel = (ii[:, None] // C) == heads[None, :]
    As_mat = jnp.where(sel, att_src.reshape(-1)[:, None], 0.0)
    Ad_mat = jnp.where(sel, att_dst.reshape(-1)[:, None], 0.0)

    hs0, hs1, hs2, hs3, ad, m = _project(x, W_gat, As_mat, Ad_mat)
    hs = jnp.concatenate([hs0, hs1, hs2, hs3], axis=0)

    src4d = edge_index[0].reshape(NTILES * NBATCH, NCHUNK, 1, CHUNK)
    dst4d = edge_index[1].reshape(NTILES * NBATCH, NCHUNK, 1, CHUNK)
    m16 = jnp.pad(m.reshape(HEADS), (0, 16 - HEADS))
    num = _sc_edge(src4d, dst4d, hs, ad, m16)
    numq = [num[q * ACC_N:q * ACC_N + N] for q in range(4)]

    return _finalize([hs0, hs1, hs2, hs3], numq, ad, m,
                     bias_gat.reshape(1, HEADS * C),
                     W1, b1.reshape(1, HID // 2), W2,
                     b2.reshape(1, NUM_CLASSES))


# speedup vs baseline: 1.1683x; 1.1683x over previous
import functools

import jax
import jax.numpy as jnp
from jax import lax
from jax.experimental import pallas as pl
from jax.experimental.pallas import tpu as pltpu
from jax.experimental.pallas import tpu_sc as plsc

N = 10000
E = 320000
D_IN = 128
HEADS = 8
C = 32
HID = 256
NUM_CLASSES = 6

Q_ROW = 80
QC = 2 * C
CHUNK = 80
NCHUNK = 5
K = CHUNK * NCHUNK
NTILES = 16
E_PER_TILE = E // NTILES
NBATCH = E_PER_TILE // K
ROWS_PER_TILE = 632
ACC_N = NTILES * ROWS_PER_TILE
ZROWS = 8

BA = 400
BC = 400


def _proj_body(x_ref, w_ref, as_ref, ad_ref,
               hs0_ref, hs1_ref, hs2_ref, hs3_ref, adout_ref, m_ref,
               ms_acc, md_acc):
    i = pl.program_id(0)
    xb = x_ref[...]
    h = jnp.dot(xb, w_ref[...], preferred_element_type=jnp.float32)
    a_s = jnp.dot(h, as_ref[...], preferred_element_type=jnp.float32)
    a_d = jnp.dot(h, ad_ref[...], preferred_element_type=jnp.float32)
    pad = jnp.zeros((BA, Q_ROW - QC - 2), jnp.float32)
    for q, hs_ref in enumerate([hs0_ref, hs1_ref, hs2_ref, hs3_ref]):
        hs_ref[...] = jnp.concatenate(
            [h[:, q * QC:(q + 1) * QC], a_s[:, 2 * q:2 * q + 2], pad], axis=1)
    adout_ref[...] = a_d
    bm_s = jnp.max(a_s, axis=0, keepdims=True)
    bm_d = jnp.max(a_d, axis=0, keepdims=True)

    @pl.when(i == 0)
    def _():
        ms_acc[...] = bm_s
        md_acc[...] = bm_d

    @pl.when(i > 0)
    def _():
        ms_acc[...] = jnp.maximum(ms_acc[...], bm_s)
        md_acc[...] = jnp.maximum(md_acc[...], bm_d)

    @pl.when(i == pl.num_programs(0) - 1)
    def _():
        m_ref[...] = ms_acc[...] + md_acc[...]


def _project(x, W_gat, As_mat, Ad_mat):
    hs_spec = pl.BlockSpec((BA, Q_ROW), lambda i: (i, 0))
    hs_shape = jax.ShapeDtypeStruct((N, Q_ROW), jnp.float32)
    return pl.pallas_call(
        _proj_body,
        grid=(N // BA,),
        in_specs=[
            pl.BlockSpec((BA, D_IN), lambda i: (i, 0)),
            pl.BlockSpec((D_IN, HEADS * C), lambda i: (0, 0)),
            pl.BlockSpec((HEADS * C, HEADS), lambda i: (0, 0)),
            pl.BlockSpec((HEADS * C, HEADS), lambda i: (0, 0)),
        ],
        out_specs=[
            hs_spec, hs_spec, hs_spec, hs_spec,
            pl.BlockSpec((BA, HEADS), lambda i: (i, 0)),
            pl.BlockSpec((1, HEADS), lambda i: (0, 0)),
        ],
        out_shape=[
            hs_shape, hs_shape, hs_shape, hs_shape,
            jax.ShapeDtypeStruct((N, HEADS), jnp.float32),
            jax.ShapeDtypeStruct((1, HEADS), jnp.float32),
        ],
        scratch_shapes=[
            pltpu.VMEM((1, HEADS), jnp.float32),
            pltpu.VMEM((1, HEADS), jnp.float32),
        ],
    )(x, W_gat, As_mat, Ad_mat)


def _splat(vec, lane):
    return lax.gather(
        vec, jnp.full((16, 1), lane, jnp.int32),
        lax.GatherDimensionNumbers(offset_dims=(), collapsed_slice_dims=(0,),
                                   start_index_map=(0,)),
        (1,), mode=lax.GatherScatterMode.PROMISE_IN_BOUNDS)


@functools.partial(
    pl.kernel,
    mesh=plsc.VectorSubcoreMesh(core_axis_name="c", subcore_axis_name="s"),
    compiler_params=pltpu.CompilerParams(use_tc_tiling_on_sc=False,
                                         needs_layout_passes=False),
    out_type=jax.ShapeDtypeStruct((4 * ACC_N, Q_ROW), jnp.float32),
    scratch_types=[
        pltpu.VMEM((2, NCHUNK, 1, CHUNK), jnp.int32),
        pltpu.VMEM((2, NCHUNK, 1, CHUNK), jnp.int32),
        pltpu.VMEM((2, NCHUNK, 1, CHUNK), jnp.int32),
        pltpu.VMEM((2, K, Q_ROW), jnp.float32),
        pltpu.VMEM((2, K, HEADS), jnp.float32),
        pltpu.VMEM((16,), jnp.float32),
        pltpu.VMEM((ZROWS, Q_ROW), jnp.float32),
        pltpu.VMEM_SHARED((ACC_N, Q_ROW), jnp.float32),
        pltpu.SemaphoreType.DMA((2, NCHUNK)),
        pltpu.SemaphoreType.DMA((2, NCHUNK)),
        pltpu.SemaphoreType.DMA((2, NCHUNK)),
    ],
)
def _sc_edge(src_hbm, dst_hbm, hs_hbm, ad_hbm, m_hbm, out_hbm,
             src_v, dst_v, srcadj_v, rows_v, adv, m_v, zbuf, acc,
             gsem, asem, ssem):
    c = lax.axis_index("c")
    sid = lax.axis_index("s")

    pltpu.sync_copy(m_hbm, m_v)

    def zrow(r, _):
        for k in range(Q_ROW // 16):
            zbuf[r, pl.ds(k * 16, 16)] = jnp.zeros((16,), jnp.float32)
        return 0

    lax.fori_loop(0, ZROWS, zrow, 0)

    iota16 = lax.iota(jnp.int32, 16)

    for half in range(2):
        q = 2 * half + c
        qn = q * N

        def zacc(j, _):
            pltpu.sync_copy(
                zbuf, acc.at[pl.ds(sid * ROWS_PER_TILE + j * ZROWS, ZROWS)])
            return 0

        lax.fori_loop(0, ROWS_PER_TILE // ZROWS, zacc, 0)
        plsc.subcore_barrier()

        def fire_batch(b, buf):
            bidx = sid * NBATCH + b
            pltpu.sync_copy(src_hbm.at[bidx], src_v.at[buf])
            pltpu.sync_copy(dst_hbm.at[bidx], dst_v.at[buf])
            for j in range(NCHUNK):
                for k in range(CHUNK // 16):
                    srcadj_v[buf, j, 0, pl.ds(k * 16, 16)] = (
                        src_v[buf, j, 0, pl.ds(k * 16, 16)] + qn)
            for j in range(NCHUNK):
                pltpu.async_copy(
                    hs_hbm.at[srcadj_v.at[buf, j, 0]],
                    rows_v.at[buf, pl.ds(j * CHUNK, CHUNK)], gsem.at[buf, j])
                pltpu.async_copy(
                    ad_hbm.at[dst_v.at[buf, j, 0]],
                    adv.at[buf, pl.ds(j * CHUNK, CHUNK)], asem.at[buf, j])

        def wait_scatters(buf):
            for j in range(NCHUNK):
                pltpu.make_async_copy(
                    rows_v.at[buf, pl.ds(j * CHUNK, CHUNK)],
                    acc.at[dst_v.at[buf, j, 0]], ssem.at[buf, j]).wait()

        def make_group(buf):
            def group(g, _):
                base = g * 16
                rid = base + iota16
                wv = []
                for hd in range(2):
                    col_w = jnp.full((16,), QC + hd, jnp.int32)
                    a_s = plsc.load_gather(rows_v.at[buf], [rid, col_w])
                    hcol = jnp.zeros((16,), jnp.int32) + (2 * q + hd)
                    a_d = plsc.load_gather(adv.at[buf], [rid, hcol])
                    v = a_s + a_d
                    lr = jnp.where(v >= 0, v, 0.2 * v)
                    w = jnp.exp(lr - plsc.load_gather(m_v, [hcol]))
                    plsc.store_scatter(rows_v.at[buf], [rid, col_w], w)
                    wv.append(w)
                for e in range(16):
                    r = base + e
                    for hd in range(2):
                        ws = _splat(wv[hd], e)
                        for hv in range(2):
                            sl = pl.ds(hd * C + hv * 16, 16)
                            rows_v[buf, r, sl] = rows_v[buf, r, sl] * ws
                return 0

            return group

        def phase(b, buf, obuf):
            @pl.when(jnp.logical_and(b >= 1, b + 1 < NBATCH))
            def _():
                wait_scatters(obuf)

            @pl.when(b + 1 < NBATCH)
            def _():
                fire_batch(b + 1, obuf)

            group = make_group(buf)
            gpc = CHUNK // 16
            for j in range(NCHUNK):
                pltpu.make_async_copy(
                    hs_hbm.at[srcadj_v.at[buf, j, 0]],
                    rows_v.at[buf, pl.ds(j * CHUNK, CHUNK)],
                    gsem.at[buf, j]).wait()
                pltpu.make_async_copy(
                    ad_hbm.at[dst_v.at[buf, j, 0]],
                    adv.at[buf, pl.ds(j * CHUNK, CHUNK)],
                    asem.at[buf, j]).wait()
                lax.fori_loop(j * gpc, (j + 1) * gpc, group, 0)
                pltpu.async_copy(
                    rows_v.at[buf, pl.ds(j * CHUNK, CHUNK)],
                    acc.at[dst_v.at[buf, j, 0]], ssem.at[buf, j], add=True)

        fire_batch(0, 0)

        def two_phases(t, _):
            phase(2 * t, 0, 1)
            phase(2 * t + 1, 1, 0)
            return 0

        lax.fori_loop(0, NBATCH // 2, two_phases, 0)
        wait_scatters(0)
        wait_scatters(1)
        plsc.subcore_barrier()
        pltpu.sync_copy(
            acc.at[pl.ds(sid * ROWS_PER_TILE, ROWS_PER_TILE)],
            out_hbm.at[pl.ds(q * ACC_N + sid * ROWS_PER_TILE,
                             ROWS_PER_TILE)])


def _final_body(hs0_ref, hs1_ref, hs2_ref, hs3_ref,
                n0_ref, n1_ref, n2_ref, n3_ref, ad_ref, m_ref,
                bias_ref, w1_ref, b1_ref, w2_ref, b2_ref, y_ref, acc):
    i = pl.program_id(0)
    ad = ad_ref[...]
    m = m_ref[...]
    ii = lax.broadcasted_iota(jnp.int32, (2, QC), 1) // C
    hh = lax.broadcasted_iota(jnp.int32, (2, QC), 0)
    expand = (ii == hh).astype(jnp.float32)

    outs = []
    for q, (hs_ref, nm_ref) in enumerate([(hs0_ref, n0_ref), (hs1_ref, n1_ref),
                                          (hs2_ref, n2_ref), (hs3_ref, n3_ref)]):
        hs = hs_ref[...]
        nm = nm_ref[...]
        h = hs[:, :QC]
        a_s = hs[:, QC:QC + 2]
        a_d = ad[:, 2 * q:2 * q + 2]
        mm = m[:, 2 * q:2 * q + 2]
        v = a_s + a_d
        lr = jnp.where(v >= 0, v, 0.2 * v)
        ws = jnp.exp(lr - mm)
        den = nm[:, QC:QC + 2] + ws
        ws_x = jnp.dot(ws, expand, preferred_element_type=jnp.float32)
        den_x = jnp.dot(den, expand, preferred_element_type=jnp.float32)
        outs.append((nm[:, :QC] + ws_x * h) / den_x)
    out = jnp.concatenate(outs, axis=1) + bias_ref[...]
    out = jnp.where(out > 0, out, jnp.exp(out) - 1.0)
    psum = jnp.sum(out, axis=0, keepdims=True)

    @pl.when(i == 0)
    def _():
        acc[...] = psum

    @pl.when(i > 0)
    def _():
        acc[...] = acc[...] + psum

    @pl.when(i == pl.num_programs(0) - 1)
    def _():
        pooled = acc[...] * (1.0 / N)
        hmid = jnp.maximum(
            jnp.dot(pooled, w1_ref[...], preferred_element_type=jnp.float32)
            + b1_ref[...], 0.0)
        y_ref[...] = (jnp.dot(hmid, w2_ref[...],
                              preferred_element_type=jnp.float32)
                      + b2_ref[...])


def _finalize(hsq, numq, ad, m, bias_gat, W1, b1, W2, b2):
    hs_spec = pl.BlockSpec((BC, Q_ROW), lambda i: (i, 0))
    return pl.pallas_call(
        _final_body,
        grid=(N // BC,),
        in_specs=[
            hs_spec, hs_spec, hs_spec, hs_spec,
            hs_spec, hs_spec, hs_spec, hs_spec,
            pl.BlockSpec((BC, HEADS), lambda i: (i, 0)),
            pl.BlockSpec((1, HEADS), lambda i: (0, 0)),
            pl.BlockSpec((1, HEADS * C), lambda i: (0, 0)),
            pl.BlockSpec((HID, HID // 2), lambda i: (0, 0)),
            pl.BlockSpec((1, HID // 2), lambda i: (0, 0)),
            pl.BlockSpec((HID // 2, NUM_CLASSES), lambda i: (0, 0)),
            pl.BlockSpec((1, NUM_CLASSES), lambda i: (0, 0)),
        ],
        out_specs=pl.BlockSpec((1, NUM_CLASSES), lambda i: (0, 0)),
        out_shape=jax.ShapeDtypeStruct((1, NUM_CLASSES), jnp.float32),
        scratch_shapes=[pltpu.VMEM((1, HEADS * C), jnp.float32)],
    )(*hsq, *numq, ad, m, bias_gat, W1, b1, W2, b2)


def kernel(x, edge_index, W_gat, att_src, att_dst, bias_gat, W1, b1, W2, b2):
    ii = jnp.arange(HEADS * C)
    heads = jnp.arange(HEADS)
    sel = (ii[:, None] // C) == heads[None, :]
    As_mat = jnp.where(sel, att_src.reshape(-1)[:, None], 0.0)
    Ad_mat = jnp.where(sel, att_dst.reshape(-1)[:, None], 0.0)

    hs0, hs1, hs2, hs3, ad, m = _project(x, W_gat, As_mat, Ad_mat)
    hs = jnp.concatenate([hs0, hs1, hs2, hs3], axis=0)

    src4d = edge_index[0].reshape(NTILES * NBATCH, NCHUNK, 1, CHUNK)
    dst4d = edge_index[1].reshape(NTILES * NBATCH, NCHUNK, 1, CHUNK)
    m16 = jnp.pad(m.reshape(HEADS), (0, 16 - HEADS))
    num = _sc_edge(src4d, dst4d, hs, ad, m16)
    numq = [num[q * ACC_N:q * ACC_N + N] for q in range(4)]

    return _finalize([hs0, hs1, hs2, hs3], numq, ad, m,
                     bias_gat.reshape(1, HEADS * C),
                     W1, b1.reshape(1, HID // 2), W2,
                     b2.reshape(1, NUM_CLASSES))


# speedup vs baseline: 1.2124x; 1.0378x over previous
import functools

import jax
import jax.numpy as jnp
from jax import lax
from jax.experimental import pallas as pl
from jax.experimental.pallas import tpu as pltpu
from jax.experimental.pallas import tpu_sc as plsc

N = 10000
E = 320000
D_IN = 128
HEADS = 8
C = 32
HID = 256
NUM_CLASSES = 6

Q_ROW = 72
QC = 2 * C
CHUNK = 80
NCHUNK = 5
K = CHUNK * NCHUNK
NTILES = 16
E_PER_TILE = E // NTILES
NBATCH = E_PER_TILE // K
ROWS_PER_TILE = 800
ACC_N = NTILES * ROWS_PER_TILE
ZROWS = 8

BA = 400
BC = 400


def _proj_body(x_ref, w_ref, as_ref, ad_ref,
               hs0_ref, hs1_ref, hs2_ref, hs3_ref, adout_ref, m_ref,
               ms_acc, md_acc):
    i = pl.program_id(0)
    xb = x_ref[...]
    h = jnp.dot(xb, w_ref[...], preferred_element_type=jnp.float32)
    a_s = jnp.dot(h, as_ref[...], preferred_element_type=jnp.float32)
    a_d = jnp.dot(h, ad_ref[...], preferred_element_type=jnp.float32)
    pad = jnp.zeros((BA, Q_ROW - QC - 2), jnp.float32)
    for q, hs_ref in enumerate([hs0_ref, hs1_ref, hs2_ref, hs3_ref]):
        hs_ref[...] = jnp.concatenate(
            [h[:, q * QC:(q + 1) * QC], a_s[:, 2 * q:2 * q + 2], pad], axis=1)
    adout_ref[...] = a_d
    bm_s = jnp.max(a_s, axis=0, keepdims=True)
    bm_d = jnp.max(a_d, axis=0, keepdims=True)

    @pl.when(i == 0)
    def _():
        ms_acc[...] = bm_s
        md_acc[...] = bm_d

    @pl.when(i > 0)
    def _():
        ms_acc[...] = jnp.maximum(ms_acc[...], bm_s)
        md_acc[...] = jnp.maximum(md_acc[...], bm_d)

    @pl.when(i == pl.num_programs(0) - 1)
    def _():
        m_ref[...] = ms_acc[...] + md_acc[...]


def _project(x, W_gat, As_mat, Ad_mat):
    hs_spec = pl.BlockSpec((BA, Q_ROW), lambda i: (i, 0))
    hs_shape = jax.ShapeDtypeStruct((N, Q_ROW), jnp.float32)
    return pl.pallas_call(
        _proj_body,
        grid=(N // BA,),
        in_specs=[
            pl.BlockSpec((BA, D_IN), lambda i: (i, 0)),
            pl.BlockSpec((D_IN, HEADS * C), lambda i: (0, 0)),
            pl.BlockSpec((HEADS * C, HEADS), lambda i: (0, 0)),
            pl.BlockSpec((HEADS * C, HEADS), lambda i: (0, 0)),
        ],
        out_specs=[
            hs_spec, hs_spec, hs_spec, hs_spec,
            pl.BlockSpec((BA, HEADS), lambda i: (i, 0)),
            pl.BlockSpec((1, HEADS), lambda i: (0, 0)),
        ],
        out_shape=[
            hs_shape, hs_shape, hs_shape, hs_shape,
            jax.ShapeDtypeStruct((N, HEADS), jnp.float32),
            jax.ShapeDtypeStruct((1, HEADS), jnp.float32),
        ],
        scratch_shapes=[
            pltpu.VMEM((1, HEADS), jnp.float32),
            pltpu.VMEM((1, HEADS), jnp.float32),
        ],
    )(x, W_gat, As_mat, Ad_mat)


def _splat(vec, lane):
    return lax.gather(
        vec, jnp.full((16, 1), lane, jnp.int32),
        lax.GatherDimensionNumbers(offset_dims=(), collapsed_slice_dims=(0,),
                                   start_index_map=(0,)),
        (1,), mode=lax.GatherScatterMode.PROMISE_IN_BOUNDS)


@functools.partial(
    pl.kernel,
    mesh=plsc.VectorSubcoreMesh(core_axis_name="c", subcore_axis_name="s"),
    compiler_params=pltpu.CompilerParams(use_tc_tiling_on_sc=False,
                                         needs_layout_passes=False),
    out_type=jax.ShapeDtypeStruct((4 * ACC_N, Q_ROW), jnp.float32),
    scratch_types=[
        pltpu.VMEM((2, NCHUNK, 1, CHUNK), jnp.int32),
        pltpu.VMEM((2, NCHUNK, 1, CHUNK), jnp.int32),
        pltpu.VMEM((2, NCHUNK, 1, CHUNK), jnp.int32),
        pltpu.VMEM((2, K, Q_ROW), jnp.float32),
        pltpu.VMEM((2, K, HEADS), jnp.float32),
        pltpu.VMEM((16,), jnp.float32),
        pltpu.VMEM((ZROWS, Q_ROW), jnp.float32),
        pltpu.VMEM_SHARED((ACC_N, Q_ROW), jnp.float32),
        pltpu.SemaphoreType.DMA((2, NCHUNK)),
        pltpu.SemaphoreType.DMA((2, NCHUNK)),
        pltpu.SemaphoreType.DMA((2, NCHUNK)),
    ],
)
def _sc_edge(src_hbm, dst_hbm, hs_hbm, ad_hbm, m_hbm, out_hbm,
             src_v, dst_v, srcadj_v, rows_v, adv, m_v, zbuf, acc,
             gsem, asem, ssem):
    c = lax.axis_index("c")
    sid = lax.axis_index("s")

    pltpu.sync_copy(m_hbm, m_v)

    def zrow(r, _):
        for k in range(Q_ROW // 16):
            zbuf[r, pl.ds(k * 16, 16)] = jnp.zeros((16,), jnp.float32)
        if Q_ROW % 16:
            zbuf[r, pl.ds(Q_ROW - 16, 16)] = jnp.zeros((16,), jnp.float32)
        return 0

    lax.fori_loop(0, ZROWS, zrow, 0)

    iota16 = lax.iota(jnp.int32, 16)

    for half in range(2):
        q = 2 * half + c
        qn = q * N

        def zacc(j, _):
            pltpu.sync_copy(
                zbuf, acc.at[pl.ds(sid * ROWS_PER_TILE + j * ZROWS, ZROWS)])
            return 0

        lax.fori_loop(0, ROWS_PER_TILE // ZROWS, zacc, 0)
        plsc.subcore_barrier()

        def fire_batch(b, buf):
            bidx = sid * NBATCH + b
            pltpu.sync_copy(src_hbm.at[bidx], src_v.at[buf])
            pltpu.sync_copy(dst_hbm.at[bidx], dst_v.at[buf])
            for j in range(NCHUNK):
                for k in range(CHUNK // 16):
                    srcadj_v[buf, j, 0, pl.ds(k * 16, 16)] = (
                        src_v[buf, j, 0, pl.ds(k * 16, 16)] + qn)
            for j in range(NCHUNK):
                pltpu.async_copy(
                    hs_hbm.at[srcadj_v.at[buf, j, 0]],
                    rows_v.at[buf, pl.ds(j * CHUNK, CHUNK)], gsem.at[buf, j])
                pltpu.async_copy(
                    ad_hbm.at[dst_v.at[buf, j, 0]],
                    adv.at[buf, pl.ds(j * CHUNK, CHUNK)], asem.at[buf, j])

        def wait_scatters(buf):
            for j in range(NCHUNK):
                pltpu.make_async_copy(
                    rows_v.at[buf, pl.ds(j * CHUNK, CHUNK)],
                    acc.at[dst_v.at[buf, j, 0]], ssem.at[buf, j]).wait()

        def make_group(buf):
            def group(g, _):
                base = g * 16
                rid = base + iota16
                wv = []
                for hd in range(2):
                    col_w = jnp.full((16,), QC + hd, jnp.int32)
                    a_s = plsc.load_gather(rows_v.at[buf], [rid, col_w])
                    hcol = jnp.zeros((16,), jnp.int32) + (2 * q + hd)
                    a_d = plsc.load_gather(adv.at[buf], [rid, hcol])
                    v = a_s + a_d
                    lr = jnp.where(v >= 0, v, 0.2 * v)
                    w = jnp.exp(lr - plsc.load_gather(m_v, [hcol]))
                    plsc.store_scatter(rows_v.at[buf], [rid, col_w], w)
                    wv.append(w)
                for e in range(16):
                    r = base + e
                    for hd in range(2):
                        ws = _splat(wv[hd], e)
                        for hv in range(2):
                            sl = pl.ds(hd * C + hv * 16, 16)
                            rows_v[buf, r, sl] = rows_v[buf, r, sl] * ws
                return 0

            return group

        def phase(b, buf, obuf):
            @pl.when(jnp.logical_and(b >= 1, b + 1 < NBATCH))
            def _():
                wait_scatters(obuf)

            @pl.when(b + 1 < NBATCH)
            def _():
                fire_batch(b + 1, obuf)

            group = make_group(buf)
            gpc = CHUNK // 16
            for j in range(NCHUNK):
                pltpu.make_async_copy(
                    hs_hbm.at[srcadj_v.at[buf, j, 0]],
                    rows_v.at[buf, pl.ds(j * CHUNK, CHUNK)],
                    gsem.at[buf, j]).wait()
                pltpu.make_async_copy(
                    ad_hbm.at[dst_v.at[buf, j, 0]],
                    adv.at[buf, pl.ds(j * CHUNK, CHUNK)],
                    asem.at[buf, j]).wait()
                lax.fori_loop(j * gpc, (j + 1) * gpc, group, 0)
                pltpu.async_copy(
                    rows_v.at[buf, pl.ds(j * CHUNK, CHUNK)],
                    acc.at[dst_v.at[buf, j, 0]], ssem.at[buf, j], add=True)

        fire_batch(0, 0)

        def two_phases(t, _):
            phase(2 * t, 0, 1)
            phase(2 * t + 1, 1, 0)
            return 0

        lax.fori_loop(0, NBATCH // 2, two_phases, 0)
        wait_scatters(0)
        wait_scatters(1)
        plsc.subcore_barrier()
        pltpu.sync_copy(
            acc.at[pl.ds(sid * ROWS_PER_TILE, ROWS_PER_TILE)],
            out_hbm.at[pl.ds(q * ACC_N + sid * ROWS_PER_TILE,
                             ROWS_PER_TILE)])


def _final_body(hs0_ref, hs1_ref, hs2_ref, hs3_ref,
                n0_ref, n1_ref, n2_ref, n3_ref, ad_ref, m_ref,
                bias_ref, w1_ref, b1_ref, w2_ref, b2_ref, y_ref, acc):
    i = pl.program_id(0)
    ad = ad_ref[...]
    m = m_ref[...]
    ii = lax.broadcasted_iota(jnp.int32, (2, QC), 1) // C
    hh = lax.broadcasted_iota(jnp.int32, (2, QC), 0)
    expand = (ii == hh).astype(jnp.float32)

    outs = []
    for q, (hs_ref, nm_ref) in enumerate([(hs0_ref, n0_ref), (hs1_ref, n1_ref),
                                          (hs2_ref, n2_ref), (hs3_ref, n3_ref)]):
        hs = hs_ref[...]
        nm = nm_ref[...]
        h = hs[:, :QC]
        a_s = hs[:, QC:QC + 2]
        a_d = ad[:, 2 * q:2 * q + 2]
        mm = m[:, 2 * q:2 * q + 2]
        v = a_s + a_d
        lr = jnp.where(v >= 0, v, 0.2 * v)
        ws = jnp.exp(lr - mm)
        den = nm[:, QC:QC + 2] + ws
        ws_x = jnp.dot(ws, expand, preferred_element_type=jnp.float32)
        den_x = jnp.dot(den, expand, preferred_element_type=jnp.float32)
        outs.append((nm[:, :QC] + ws_x * h) / den_x)
    out = jnp.concatenate(outs, axis=1) + bias_ref[...]
    out = jnp.where(out > 0, out, jnp.exp(out) - 1.0)
    psum = jnp.sum(out, axis=0, keepdims=True)

    @pl.when(i == 0)
    def _():
        acc[...] = psum

    @pl.when(i > 0)
    def _():
        acc[...] = acc[...] + psum

    @pl.when(i == pl.num_programs(0) - 1)
    def _():
        pooled = acc[...] * (1.0 / N)
        hmid = jnp.maximum(
            jnp.dot(pooled, w1_ref[...], preferred_element_type=jnp.float32)
            + b1_ref[...], 0.0)
        y_ref[...] = (jnp.dot(hmid, w2_ref[...],
                              preferred_element_type=jnp.float32)
                      + b2_ref[...])


def _finalize(hsq, num, ad, m, bias_gat, W1, b1, W2, b2):
    hs_spec = pl.BlockSpec((BC, Q_ROW), lambda i: (i, 0))
    nblk = ACC_N // BC
    num_specs = [
        pl.BlockSpec((BC, Q_ROW), lambda i, q=q: (q * nblk + i, 0))
        for q in range(4)
    ]
    return pl.pallas_call(
        _final_body,
        grid=(N // BC,),
        in_specs=[
            hs_spec, hs_spec, hs_spec, hs_spec,
            *num_specs,
            pl.BlockSpec((BC, HEADS), lambda i: (i, 0)),
            pl.BlockSpec((1, HEADS), lambda i: (0, 0)),
            pl.BlockSpec((1, HEADS * C), lambda i: (0, 0)),
            pl.BlockSpec((HID, HID // 2), lambda i: (0, 0)),
            pl.BlockSpec((1, HID // 2), lambda i: (0, 0)),
            pl.BlockSpec((HID // 2, NUM_CLASSES), lambda i: (0, 0)),
            pl.BlockSpec((1, NUM_CLASSES), lambda i: (0, 0)),
        ],
        out_specs=pl.BlockSpec((1, NUM_CLASSES), lambda i: (0, 0)),
        out_shape=jax.ShapeDtypeStruct((1, NUM_CLASSES), jnp.float32),
        scratch_shapes=[pltpu.VMEM((1, HEADS * C), jnp.float32)],
    )(*hsq, num, num, num, num, ad, m, bias_gat, W1, b1, W2, b2)


def kernel(x, edge_index, W_gat, att_src, att_dst, bias_gat, W1, b1, W2, b2):
    ii = jnp.arange(HEADS * C)
    heads = jnp.arange(HEADS)
    sel = (ii[:, None] // C) == heads[None, :]
    As_mat = jnp.where(sel, att_src.reshape(-1)[:, None], 0.0)
    Ad_mat = jnp.where(sel, att_dst.reshape(-1)[:, None], 0.0)

    hs0, hs1, hs2, hs3, ad, m = _project(x, W_gat, As_mat, Ad_mat)
    hs = jnp.concatenate([hs0, hs1, hs2, hs3], axis=0)

    src4d = edge_index[0].reshape(NTILES * NBATCH, NCHUNK, 1, CHUNK)
    dst4d = edge_index[1].reshape(NTILES * NBATCH, NCHUNK, 1, CHUNK)
    m16 = jnp.pad(m.reshape(HEADS), (0, 16 - HEADS))
    num = _sc_edge(src4d, dst4d, hs, ad, m16)

    return _finalize([hs0, hs1, hs2, hs3], num, ad, m,
                     bias_gat.reshape(1, HEADS * C),
                     W1, b1.reshape(1, HID // 2), W2,
                     b2.reshape(1, NUM_CLASSES))


# speedup vs baseline: 1.3256x; 1.0933x over previous
import functools

import jax
import jax.numpy as jnp
from jax import lax
from jax.experimental import pallas as pl
from jax.experimental.pallas import tpu as pltpu
from jax.experimental.pallas import tpu_sc as plsc

N = 10000
E = 320000
D_IN = 128
HEADS = 8
C = 32
HID = 256
NUM_CLASSES = 6

Q_ROW = 72
QC = 2 * C
CHUNK = 80
NCHUNK = 5
K = CHUNK * NCHUNK
NTILES = 16
E_PER_TILE = E // NTILES
NBATCH = E_PER_TILE // K
ROWS_PER_TILE = 800
ACC_N = NTILES * ROWS_PER_TILE
ZROWS = 8

BA = 400
BC = 400


def _proj_body(x_ref, w_ref, as_ref, ad_ref,
               hs0_ref, hs1_ref, hs2_ref, hs3_ref, adout_ref, m_ref,
               ms_acc, md_acc):
    i = pl.program_id(0)
    xb = x_ref[...]
    h = jnp.dot(xb, w_ref[...], preferred_element_type=jnp.float32)
    a_s = jnp.dot(h, as_ref[...], preferred_element_type=jnp.float32)
    a_d = jnp.dot(h, ad_ref[...], preferred_element_type=jnp.float32)
    pad = jnp.zeros((BA, Q_ROW - QC - 2), jnp.float32)
    for q, hs_ref in enumerate([hs0_ref, hs1_ref, hs2_ref, hs3_ref]):
        hs_ref[...] = jnp.concatenate(
            [h[:, q * QC:(q + 1) * QC], a_s[:, 2 * q:2 * q + 2], pad], axis=1)
    adout_ref[...] = a_d
    bm_s = jnp.max(a_s, axis=0, keepdims=True)
    bm_d = jnp.max(a_d, axis=0, keepdims=True)

    @pl.when(i == 0)
    def _():
        ms_acc[...] = bm_s
        md_acc[...] = bm_d

    @pl.when(i > 0)
    def _():
        ms_acc[...] = jnp.maximum(ms_acc[...], bm_s)
        md_acc[...] = jnp.maximum(md_acc[...], bm_d)

    @pl.when(i == pl.num_programs(0) - 1)
    def _():
        m_ref[...] = ms_acc[...] + md_acc[...]


def _project(x, W_gat, As_mat, Ad_mat):
    hs_spec = pl.BlockSpec((BA, Q_ROW), lambda i: (i, 0))
    hs_shape = jax.ShapeDtypeStruct((N, Q_ROW), jnp.float32)
    return pl.pallas_call(
        _proj_body,
        grid=(N // BA,),
        in_specs=[
            pl.BlockSpec((BA, D_IN), lambda i: (i, 0)),
            pl.BlockSpec((D_IN, HEADS * C), lambda i: (0, 0)),
            pl.BlockSpec((HEADS * C, HEADS), lambda i: (0, 0)),
            pl.BlockSpec((HEADS * C, HEADS), lambda i: (0, 0)),
        ],
        out_specs=[
            hs_spec, hs_spec, hs_spec, hs_spec,
            pl.BlockSpec((BA, HEADS), lambda i: (i, 0)),
            pl.BlockSpec((1, HEADS), lambda i: (0, 0)),
        ],
        out_shape=[
            hs_shape, hs_shape, hs_shape, hs_shape,
            jax.ShapeDtypeStruct((N, HEADS), jnp.float32),
            jax.ShapeDtypeStruct((1, HEADS), jnp.float32),
        ],
        scratch_shapes=[
            pltpu.VMEM((1, HEADS), jnp.float32),
            pltpu.VMEM((1, HEADS), jnp.float32),
        ],
    )(x, W_gat, As_mat, Ad_mat)


def _splat(vec, lane):
    return lax.gather(
        vec, jnp.full((16, 1), lane, jnp.int32),
        lax.GatherDimensionNumbers(offset_dims=(), collapsed_slice_dims=(0,),
                                   start_index_map=(0,)),
        (1,), mode=lax.GatherScatterMode.PROMISE_IN_BOUNDS)


@functools.partial(
    pl.kernel,
    mesh=plsc.VectorSubcoreMesh(core_axis_name="c", subcore_axis_name="s"),
    compiler_params=pltpu.CompilerParams(use_tc_tiling_on_sc=False,
                                         needs_layout_passes=False),
    out_type=jax.ShapeDtypeStruct((4 * ACC_N, Q_ROW), jnp.float32),
    scratch_types=[
        pltpu.VMEM((2, NCHUNK, 1, CHUNK), jnp.int32),
        pltpu.VMEM((2, NCHUNK, 1, CHUNK), jnp.int32),
        pltpu.VMEM((2, NCHUNK, 1, CHUNK), jnp.int32),
        pltpu.VMEM((2, K, Q_ROW), jnp.float32),
        pltpu.VMEM((2, K, HEADS), jnp.float32),
        pltpu.VMEM((16,), jnp.float32),
        pltpu.VMEM((ZROWS, Q_ROW), jnp.float32),
        pltpu.VMEM_SHARED((ACC_N, Q_ROW), jnp.float32),
        pltpu.SemaphoreType.DMA((2, NCHUNK)),
        pltpu.SemaphoreType.DMA((2, NCHUNK)),
        pltpu.SemaphoreType.DMA((2, NCHUNK)),
    ],
)
def _sc_edge(e_hbm, hs_hbm, ad_hbm, m_hbm, out_hbm,
             ev, dst_v, srcadj_v, rows_v, adv, m_v, zbuf, acc,
             gsem, asem, ssem):
    c = lax.axis_index("c")
    sid = lax.axis_index("s")

    pltpu.sync_copy(m_hbm, m_v)

    def zrow(r, _):
        for k in range(Q_ROW // 16):
            zbuf[r, pl.ds(k * 16, 16)] = jnp.zeros((16,), jnp.float32)
        if Q_ROW % 16:
            zbuf[r, pl.ds(Q_ROW - 16, 16)] = jnp.zeros((16,), jnp.float32)
        return 0

    lax.fori_loop(0, ZROWS, zrow, 0)

    iota16 = lax.iota(jnp.int32, 16)

    for half in range(2):
        q = 2 * half + c
        qn = q * N

        def zacc(j, _):
            pltpu.sync_copy(
                zbuf, acc.at[pl.ds(sid * ROWS_PER_TILE + j * ZROWS, ZROWS)])
            return 0

        lax.fori_loop(0, ROWS_PER_TILE // ZROWS, zacc, 0)
        plsc.subcore_barrier()

        def fire_batch(b, buf):
            bidx = sid * NBATCH + b
            pltpu.sync_copy(e_hbm.at[bidx], ev.at[buf])
            for j in range(NCHUNK):
                for k in range(CHUNK // 16):
                    sl = pl.ds(k * 16, 16)
                    val = ev[buf, j, 0, sl]
                    dst_v[buf, j, 0, sl] = lax.shift_right_logical(val, 14)
                    srcadj_v[buf, j, 0, sl] = (val & 16383) + qn
            for j in range(NCHUNK):
                pltpu.async_copy(
                    hs_hbm.at[srcadj_v.at[buf, j, 0]],
                    rows_v.at[buf, pl.ds(j * CHUNK, CHUNK)], gsem.at[buf, j])
                pltpu.async_copy(
                    ad_hbm.at[dst_v.at[buf, j, 0]],
                    adv.at[buf, pl.ds(j * CHUNK, CHUNK)], asem.at[buf, j])

        def wait_scatters(buf):
            for j in range(NCHUNK):
                pltpu.make_async_copy(
                    rows_v.at[buf, pl.ds(j * CHUNK, CHUNK)],
                    acc.at[dst_v.at[buf, j, 0]], ssem.at[buf, j]).wait()

        def make_group(buf):
            def group(g, _):
                base = g * 16
                rid = base + iota16
                wv = []
                for hd in range(2):
                    col_w = jnp.full((16,), QC + hd, jnp.int32)
                    a_s = plsc.load_gather(rows_v.at[buf], [rid, col_w])
                    hcol = jnp.zeros((16,), jnp.int32) + (2 * q + hd)
                    a_d = plsc.load_gather(adv.at[buf], [rid, hcol])
                    v = a_s + a_d
                    lr = jnp.where(v >= 0, v, 0.2 * v)
                    w = jnp.exp(lr - plsc.load_gather(m_v, [hcol]))
                    plsc.store_scatter(rows_v.at[buf], [rid, col_w], w)
                    wv.append(w)
                for e in range(16):
                    r = base + e
                    for hd in range(2):
                        ws = _splat(wv[hd], e)
                        for hv in range(2):
                            sl = pl.ds(hd * C + hv * 16, 16)
                            rows_v[buf, r, sl] = rows_v[buf, r, sl] * ws
                return 0

            return group

        def phase(b, buf, obuf):
            @pl.when(jnp.logical_and(b >= 1, b + 1 < NBATCH))
            def _():
                wait_scatters(obuf)

            @pl.when(b + 1 < NBATCH)
            def _():
                fire_batch(b + 1, obuf)

            group = make_group(buf)
            gpc = CHUNK // 16
            for j in range(NCHUNK):
                pltpu.make_async_copy(
                    hs_hbm.at[srcadj_v.at[buf, j, 0]],
                    rows_v.at[buf, pl.ds(j * CHUNK, CHUNK)],
                    gsem.at[buf, j]).wait()
                pltpu.make_async_copy(
                    ad_hbm.at[dst_v.at[buf, j, 0]],
                    adv.at[buf, pl.ds(j * CHUNK, CHUNK)],
                    asem.at[buf, j]).wait()
                lax.fori_loop(j * gpc, (j + 1) * gpc, group, 0)
                pltpu.async_copy(
                    rows_v.at[buf, pl.ds(j * CHUNK, CHUNK)],
                    acc.at[dst_v.at[buf, j, 0]], ssem.at[buf, j], add=True)

        fire_batch(0, 0)

        def two_phases(t, _):
            phase(2 * t, 0, 1)
            phase(2 * t + 1, 1, 0)
            return 0

        lax.fori_loop(0, NBATCH // 2, two_phases, 0)
        wait_scatters(0)
        wait_scatters(1)
        plsc.subcore_barrier()
        pltpu.sync_copy(
            acc.at[pl.ds(sid * ROWS_PER_TILE, ROWS_PER_TILE)],
            out_hbm.at[pl.ds(q * ACC_N + sid * ROWS_PER_TILE,
                             ROWS_PER_TILE)])


def _final_body(hs0_ref, hs1_ref, hs2_ref, hs3_ref,
                n0_ref, n1_ref, n2_ref, n3_ref, ad_ref, m_ref,
                bias_ref, w1_ref, b1_ref, w2_ref, b2_ref, y_ref, acc):
    i = pl.program_id(0)
    ad = ad_ref[...]
    m = m_ref[...]
    ii = lax.broadcasted_iota(jnp.int32, (2, QC), 1) // C
    hh = lax.broadcasted_iota(jnp.int32, (2, QC), 0)
    expand = (ii == hh).astype(jnp.float32)

    outs = []
    for q, (hs_ref, nm_ref) in enumerate([(hs0_ref, n0_ref), (hs1_ref, n1_ref),
                                          (hs2_ref, n2_ref), (hs3_ref, n3_ref)]):
        hs = hs_ref[...]
        nm = nm_ref[...]
        h = hs[:, :QC]
        a_s = hs[:, QC:QC + 2]
        a_d = ad[:, 2 * q:2 * q + 2]
        mm = m[:, 2 * q:2 * q + 2]
        v = a_s + a_d
        lr = jnp.where(v >= 0, v, 0.2 * v)
        ws = jnp.exp(lr - mm)
        den = nm[:, QC:QC + 2] + ws
        ws_x = jnp.dot(ws, expand, preferred_element_type=jnp.float32)
        den_x = jnp.dot(den, expand, preferred_element_type=jnp.float32)
        outs.append((nm[:, :QC] + ws_x * h) / den_x)
    out = jnp.concatenate(outs, axis=1) + bias_ref[...]
    out = jnp.where(out > 0, out, jnp.exp(out) - 1.0)
    psum = jnp.sum(out, axis=0, keepdims=True)

    @pl.when(i == 0)
    def _():
        acc[...] = psum

    @pl.when(i > 0)
    def _():
        acc[...] = acc[...] + psum

    @pl.when(i == pl.num_programs(0) - 1)
    def _():
        pooled = acc[...] * (1.0 / N)
        hmid = jnp.maximum(
            jnp.dot(pooled, w1_ref[...], preferred_element_type=jnp.float32)
            + b1_ref[...], 0.0)
        y_ref[...] = (jnp.dot(hmid, w2_ref[...],
                              preferred_element_type=jnp.float32)
                      + b2_ref[...])


def _finalize(hsq, num, ad, m, bias_gat, W1, b1, W2, b2):
    hs_spec = pl.BlockSpec((BC, Q_ROW), lambda i: (i, 0))
    nblk = ACC_N // BC
    num_specs = [
        pl.BlockSpec((BC, Q_ROW), lambda i, q=q: (q * nblk + i, 0))
        for q in range(4)
    ]
    return pl.pallas_call(
        _final_body,
        grid=(N // BC,),
        in_specs=[
            hs_spec, hs_spec, hs_spec, hs_spec,
            *num_specs,
            pl.BlockSpec((BC, HEADS), lambda i: (i, 0)),
            pl.BlockSpec((1, HEADS), lambda i: (0, 0)),
            pl.BlockSpec((1, HEADS * C), lambda i: (0, 0)),
            pl.BlockSpec((HID, HID // 2), lambda i: (0, 0)),
            pl.BlockSpec((1, HID // 2), lambda i: (0, 0)),
            pl.BlockSpec((HID // 2, NUM_CLASSES), lambda i: (0, 0)),
            pl.BlockSpec((1, NUM_CLASSES), lambda i: (0, 0)),
        ],
        out_specs=pl.BlockSpec((1, NUM_CLASSES), lambda i: (0, 0)),
        out_shape=jax.ShapeDtypeStruct((1, NUM_CLASSES), jnp.float32),
        scratch_shapes=[pltpu.VMEM((1, HEADS * C), jnp.float32)],
    )(*hsq, num, num, num, num, ad, m, bias_gat, W1, b1, W2, b2)


def kernel(x, edge_index, W_gat, att_src, att_dst, bias_gat, W1, b1, W2, b2):
    ii = jnp.arange(HEADS * C)
    heads = jnp.arange(HEADS)
    sel = (ii[:, None] // C) == heads[None, :]
    As_mat = jnp.where(sel, att_src.reshape(-1)[:, None], 0.0)
    Ad_mat = jnp.where(sel, att_dst.reshape(-1)[:, None], 0.0)

    hs0, hs1, hs2, hs3, ad, m = _project(x, W_gat, As_mat, Ad_mat)
    hs = jnp.concatenate([hs0, hs1, hs2, hs3], axis=0)

    epack = (edge_index[0] + edge_index[1] * 16384).reshape(
        NTILES * NBATCH, NCHUNK, 1, CHUNK)
    m16 = jnp.pad(m.reshape(HEADS), (0, 16 - HEADS))
    num = _sc_edge(epack, hs, ad, m16)

    return _finalize([hs0, hs1, hs2, hs3], num, ad, m,
                     bias_gat.reshape(1, HEADS * C),
                     W1, b1.reshape(1, HID // 2), W2,
                     b2.reshape(1, NUM_CLASSES))
